# Initial kernel scaffold; baseline (speedup 1.0000x reference)
import jax
import jax.numpy as jnp
from jax.experimental import pallas as pl


def kernel(query_embeddings, context_embeddings, W_q, b_q):
    raise NotImplementedError("write your pallas kernel here")


# speedup vs baseline: 6.8103x; 6.8103x over previous
import functools

import jax
import jax.numpy as jnp
from jax import lax
from jax.experimental import pallas as pl
from jax.experimental.pallas import tpu as pltpu

TOPK = 8


def _proj_body(q_ref, w_ref, b_ref, out_ref):
    q = q_ref[...]
    w = w_ref[...]
    acc = lax.dot_general(q, w, (((1,), (1,)), ((), ())),
                          preferred_element_type=jnp.float32)
    out_ref[...] = acc + b_ref[...]


def _dist_topk_body(qp_ref, ctx_ref, val_ref, idx_ref):
    qb = qp_ref[0]
    cb = ctx_ref[0]
    cross = lax.dot_general(qb, cb, (((1,), (1,)), ((), ())),
                            preferred_element_type=jnp.float32)
    q_sq = jnp.sum(qb * qb, axis=1, keepdims=True)
    c_sq = jnp.sum(cb * cb, axis=1)[None, :]
    d2 = q_sq + c_sq - 2.0 * cross
    dist = jnp.sqrt(jnp.maximum(d2, 1e-12))

    Q, C = dist.shape
    col = lax.broadcasted_iota(jnp.int32, (Q, C), 1)
    vals = []
    idxs = []
    for _ in range(TOPK):
        m = jnp.min(dist, axis=1, keepdims=True)
        hit = dist == m
        amin = jnp.min(jnp.where(hit, col, C), axis=1, keepdims=True)
        vals.append(m)
        idxs.append(amin)
        dist = jnp.where(col == amin, jnp.inf, dist)
    val_ref[0] = jnp.concatenate(vals, axis=1)
    idx_ref[0] = jnp.concatenate(idxs, axis=1)


@jax.jit
def kernel(query_embeddings, context_embeddings, W_q, b_q):
    B, Q, D = query_embeddings.shape
    C = context_embeddings.shape[1]

    q2 = query_embeddings.reshape(B * Q, D)
    qp = pl.pallas_call(
        _proj_body,
        out_shape=jax.ShapeDtypeStruct((B * Q, D), jnp.float32),
    )(q2, W_q, b_q.reshape(1, D))
    qp3 = qp.reshape(B, Q, D)

    grid = (B,)
    val, idx = pl.pallas_call(
        _dist_topk_body,
        grid=grid,
        in_specs=[
            pl.BlockSpec((1, Q, D), lambda b: (b, 0, 0)),
            pl.BlockSpec((1, C, D), lambda b: (b, 0, 0)),
        ],
        out_specs=[
            pl.BlockSpec((1, Q, TOPK), lambda b: (b, 0, 0)),
            pl.BlockSpec((1, Q, TOPK), lambda b: (b, 0, 0)),
        ],
        out_shape=[
            jax.ShapeDtypeStruct((B, Q, TOPK), jnp.float32),
            jax.ShapeDtypeStruct((B, Q, TOPK), jnp.int32),
        ],
    )(qp3, context_embeddings)
    return val, idx


# speedup vs baseline: 8.4338x; 1.2384x over previous
import functools

import jax
import jax.numpy as jnp
from jax import lax
from jax.experimental import pallas as pl
from jax.experimental.pallas import tpu as pltpu

TOPK = 8


def _proj_body(q_ref, w_ref, b_ref, out_ref):
    q = q_ref[...]
    w = w_ref[...]
    acc = lax.dot_general(q, w, (((1,), (1,)), ((), ())),
                          preferred_element_type=jnp.float32)
    out_ref[...] = acc + b_ref[...]


def _dist_topk_body(qp_ref, ctx_ref, val_ref, idx_ref):
    nb = qp_ref.shape[0]
    for b in range(nb):
        qb = qp_ref[b]
        cb = ctx_ref[b]
        cross = lax.dot_general(qb, cb, (((1,), (1,)), ((), ())),
                                preferred_element_type=jnp.float32)
        q_sq = jnp.sum(qb * qb, axis=1, keepdims=True)
        c_sq = jnp.sum(cb * cb, axis=1)[None, :]
        d2 = q_sq + c_sq - 2.0 * cross
        dist = jnp.sqrt(jnp.maximum(d2, 1e-12))

        Q, C = dist.shape
        col = lax.broadcasted_iota(jnp.int32, (Q, C), 1)
        vals = []
        idxs = []
        for _ in range(TOPK):
            m = jnp.min(dist, axis=1, keepdims=True)
            hit = dist == m
            amin = jnp.min(jnp.where(hit, col, C), axis=1, keepdims=True)
            vals.append(m)
            idxs.append(amin)
            dist = jnp.where(col == amin, jnp.inf, dist)
        val_ref[b] = jnp.concatenate(vals, axis=1)
        idx_ref[b] = jnp.concatenate(idxs, axis=1)


@jax.jit
def kernel(query_embeddings, context_embeddings, W_q, b_q):
    B, Q, D = query_embeddings.shape
    C = context_embeddings.shape[1]

    q2 = query_embeddings.reshape(B * Q, D)
    qp = pl.pallas_call(
        _proj_body,
        out_shape=jax.ShapeDtypeStruct((B * Q, D), jnp.float32),
    )(q2, W_q, b_q.reshape(1, D))
    qp3 = qp.reshape(B, Q, D)

    NB = 2
    grid = (B // NB,)
    val, idx = pl.pallas_call(
        _dist_topk_body,
        grid=grid,
        in_specs=[
            pl.BlockSpec((NB, Q, D), lambda b: (b, 0, 0)),
            pl.BlockSpec((NB, C, D), lambda b: (b, 0, 0)),
        ],
        out_specs=[
            pl.BlockSpec((NB, Q, TOPK), lambda b: (b, 0, 0)),
            pl.BlockSpec((NB, Q, TOPK), lambda b: (b, 0, 0)),
        ],
        out_shape=[
            jax.ShapeDtypeStruct((B, Q, TOPK), jnp.float32),
            jax.ShapeDtypeStruct((B, Q, TOPK), jnp.int32),
        ],
    )(qp3, context_embeddings)
    return val, idx
